# baseline (speedup 1.0000x reference)
import functools

import jax
import jax.numpy as jnp
from jax import lax
from jax.experimental import pallas as pl
from jax.experimental.pallas import tpu as pltpu
from jax.experimental.pallas import tpu_sc as plsc

_B = 16384
_F = 26
_D = 32
_N = _B * _F
_NC = 2
_NS = 16
_NW = _NC * _NS
_PER_W = _N // _NW
_CH = 128
_NCH = _PER_W // _CH
_LANES = 16
_NBUF = 4


def _body(ids_hbm, vals_hbm, table_hbm, bias_hbm, out_hbm,
          idx_v, val_v, rows_v, bias_v, out_v, sem_t, sem_b, sem_o):
    wid = lax.axis_index("s") * _NC + lax.axis_index("c")

    pltpu.sync_copy(ids_hbm.at[wid], idx_v)
    pltpu.sync_copy(vals_hbm.at[pl.ds(wid * _PER_W, _PER_W)], val_v)

    zero16 = lax.broadcasted_iota(jnp.int32, (_LANES,), 0) * 0

    def start_gathers(ch, b):
        pltpu.async_copy(table_hbm.at[idx_v.at[ch]], rows_v.at[b], sem_t.at[b])
        pltpu.async_copy(bias_hbm.at[idx_v.at[ch]], bias_v.at[b], sem_b.at[b])

    def wait_gathers(ch, b):
        pltpu.make_async_copy(
            table_hbm.at[idx_v.at[ch]], rows_v.at[b], sem_t.at[b]).wait()
        pltpu.make_async_copy(
            bias_hbm.at[idx_v.at[ch]], bias_v.at[b], sem_b.at[b]).wait()

    def out_slice(ch):
        return out_hbm.at[pl.ds((wid * _NCH + ch) * _CH, _CH)]

    def compute(ch, b):
        def grp_body(g, carry):
            val16 = val_v[pl.ds(ch * _CH + g * _LANES, _LANES)]
            for j in range(_LANES):
                row = g * _LANES + j
                vexp = val16.at[zero16 + j].get(mode="promise_in_bounds")
                r0 = rows_v[b, row, pl.ds(0, _LANES)]
                r1 = rows_v[b, row, pl.ds(_LANES, _LANES)]
                b0 = bias_v[b, row, pl.ds(0, _LANES)]
                b1 = bias_v[b, row, pl.ds(_LANES, _LANES)]
                out_v[b, row, pl.ds(0, _LANES)] = r0 * vexp + b0
                out_v[b, row, pl.ds(_LANES, _LANES)] = r1 * vexp + b1
            return carry

        lax.fori_loop(0, _CH // _LANES, grp_body, 0)

    for b in range(_NBUF):
        start_gathers(b, b)

    @pl.loop(0, _NCH, step=_NBUF)
    def _chunk_loop(c0):
        for b in range(_NBUF):
            ch = c0 + b

            @pl.when(ch >= _NBUF)
            def _():
                pltpu.make_async_copy(
                    out_v.at[b], out_slice(ch - _NBUF), sem_o.at[b]).wait()

            wait_gathers(ch, b)
            compute(ch, b)
            pltpu.async_copy(out_v.at[b], out_slice(ch), sem_o.at[b])

            @pl.when(ch + _NBUF < _NCH)
            def _():
                start_gathers(ch + _NBUF, b)

    for b in range(_NBUF):
        pltpu.make_async_copy(
            out_v.at[b], out_slice(_NCH - _NBUF + b), sem_o.at[b]).wait()


_V = 1000001
_BT = 2048


def _tp_body(inT_ref, out_ref):
    row = lax.broadcasted_iota(jnp.int32, (_D, _D), 0)
    col = lax.broadcasted_iota(jnp.int32, (_D, _D), 1)
    ident = jnp.where(row == col, 1.0, 0.0).astype(jnp.float32)
    out_ref[...] = lax.dot_general(
        inT_ref[...], ident,
        dimension_numbers=(((0,), (0,)), ((), ())),
        preferred_element_type=jnp.float32)


def _transpose_tc(tT):
    grid = (pl.cdiv(_V, _BT),)
    return pl.pallas_call(
        _tp_body,
        grid=grid,
        in_specs=[pl.BlockSpec((_D, _BT), lambda i: (0, i))],
        out_specs=pl.BlockSpec((_BT, _D), lambda i: (i, 0)),
        out_shape=jax.ShapeDtypeStruct((_V, _D), jnp.float32),
    )(tT)


@jax.jit
def _emb(ids3, vals1, table, bias_table):
    mesh = plsc.VectorSubcoreMesh(core_axis_name="c", subcore_axis_name="s")
    f = functools.partial(
        pl.kernel,
        out_type=jax.ShapeDtypeStruct((_N, _D), jnp.float32),
        mesh=mesh,
        compiler_params=pltpu.CompilerParams(use_tc_tiling_on_sc=False),
        scratch_types=[
            pltpu.VMEM((_NCH, _CH), jnp.int32),
            pltpu.VMEM((_PER_W,), jnp.float32),
            pltpu.VMEM((_NBUF, _CH, _D), jnp.float32),
            pltpu.VMEM((_NBUF, _CH, _D), jnp.float32),
            pltpu.VMEM((_NBUF, _CH, _D), jnp.float32),
            pltpu.SemaphoreType.DMA((_NBUF,)),
            pltpu.SemaphoreType.DMA((_NBUF,)),
            pltpu.SemaphoreType.DMA((_NBUF,)),
        ],
    )(_body)
    return f(ids3, vals1, table, bias_table)


def kernel(ids, values, table, bias_table):
    ids3 = ids.reshape(_NW, _NCH, _CH)
    vals1 = values.reshape(_N)
    table_l = _transpose_tc(table.T)
    out = _emb(ids3, vals1, table_l, bias_table)
    return out.reshape(_B, _F, _D)


# speedup vs baseline: 1.2205x; 1.2205x over previous
import functools

import jax
import jax.numpy as jnp
from jax import lax
from jax.experimental import pallas as pl
from jax.experimental.pallas import tpu as pltpu
from jax.experimental.pallas import tpu_sc as plsc

_B = 16384
_F = 26
_D = 32
_N = _B * _F
_NC = 2
_NS = 16
_NW = _NC * _NS
_BPW = _B // _NW
_PER_W = _N // _NW
_BCH = 8
_CH = _BCH * _F
_HCH = _CH // 2
_NCH = _BPW // _BCH
_LANES = 16
_NGRP = _CH // _LANES
_NBUF = 2
_VS = 1000000


def _body(ids_hbm, vals_hbm, table_hbm, bias_hbm, out_hbm,
          idx_v, val_v, rows_v, bias_v, out_v, sem_t, sem_b, sem_o):
    wid = lax.axis_index("s") * _NC + lax.axis_index("c")

    pltpu.sync_copy(ids_hbm.at[wid], idx_v)
    pltpu.sync_copy(vals_hbm.at[pl.ds(wid * _PER_W, _PER_W)], val_v)

    zero16 = lax.broadcasted_iota(jnp.int32, (_LANES,), 0) * 0

    def start_gathers(ch, b):
        for h in range(2):
            pltpu.async_copy(table_hbm.at[idx_v.at[ch, h]],
                             rows_v.at[b, pl.ds(h * _HCH, _HCH)], sem_t.at[b])
            pltpu.async_copy(bias_hbm.at[idx_v.at[ch, h]],
                             bias_v.at[b, pl.ds(h * _HCH, _HCH)], sem_b.at[b])

    def wait_gathers(ch, b):
        for h in range(2):
            pltpu.make_async_copy(
                table_hbm.at[idx_v.at[ch, h]],
                rows_v.at[b, pl.ds(h * _HCH, _HCH)], sem_t.at[b]).wait()
            pltpu.make_async_copy(
                bias_hbm.at[idx_v.at[ch, h]],
                bias_v.at[b, pl.ds(h * _HCH, _HCH)], sem_b.at[b]).wait()

    def out_slice(ch):
        return out_hbm.at[pl.ds(wid * _BPW + ch * _BCH, _BCH)]

    def compute(ch, b):
        for g in range(_NGRP):
            val16 = val_v[pl.ds(ch * _CH + g * _LANES, _LANES)]
            for j in range(_LANES):
                r = g * _LANES + j
                bi, fi = r // _F, r % _F
                vexp = val16.at[zero16 + j].get(mode="promise_in_bounds")
                r0 = rows_v[b, r, pl.ds(0, _LANES)]
                r1 = rows_v[b, r, pl.ds(_LANES, _LANES)]
                b0 = bias_v[b, r, pl.ds(0, _LANES)]
                b1 = bias_v[b, r, pl.ds(_LANES, _LANES)]
                out_v[b, bi, fi, pl.ds(0, _LANES)] = r0 * vexp + b0
                out_v[b, bi, fi, pl.ds(_LANES, _LANES)] = r1 * vexp + b1

    for b in range(_NBUF):
        start_gathers(b, b)

    @pl.loop(0, _NCH, step=_NBUF)
    def _chunk_loop(c0):
        for b in range(_NBUF):
            ch = c0 + b

            @pl.when(ch >= _NBUF)
            def _():
                pltpu.make_async_copy(
                    out_v.at[b], out_slice(ch - _NBUF), sem_o.at[b]).wait()

            wait_gathers(ch, b)
            compute(ch, b)
            pltpu.async_copy(out_v.at[b], out_slice(ch), sem_o.at[b])

            @pl.when(ch + _NBUF < _NCH)
            def _():
                start_gathers(ch + _NBUF, b)

    for b in range(_NBUF):
        pltpu.make_async_copy(
            out_v.at[b], out_slice(_NCH - _NBUF + b), sem_o.at[b]).wait()


@jax.jit
def _emb(ids3, vals1, table, bias_table):
    mesh = plsc.VectorSubcoreMesh(core_axis_name="c", subcore_axis_name="s")
    f = functools.partial(
        pl.kernel,
        out_type=jax.ShapeDtypeStruct((_B, _F, _D), jnp.float32),
        mesh=mesh,
        compiler_params=pltpu.CompilerParams(use_tc_tiling_on_sc=False),
        scratch_types=[
            pltpu.VMEM((_NCH, 2, _HCH), jnp.int32),
            pltpu.VMEM((_PER_W,), jnp.float32),
            pltpu.VMEM((_NBUF, _CH, _D), jnp.float32),
            pltpu.VMEM((_NBUF, _CH, _D), jnp.float32),
            pltpu.VMEM((_NBUF, _BCH, _F, _D), jnp.float32),
            pltpu.SemaphoreType.DMA((_NBUF,)),
            pltpu.SemaphoreType.DMA((_NBUF,)),
            pltpu.SemaphoreType.DMA((_NBUF,)),
        ],
    )(_body)
    return f(ids3, vals1, table, bias_table)


def kernel(ids, values, table, bias_table):
    ids3 = ids.reshape(_NW, _NCH, 2, _HCH)
    vals1 = values.reshape(_N)
    return _emb(ids3, vals1, table[:_VS], bias_table[:_VS])
